# baseline (speedup 1.0000x reference)
import jax
import jax.numpy as jnp
from jax.experimental import pallas as pl
from jax.experimental.pallas import tpu as pltpu
from jax.experimental.pallas import tpu_sc as plsc

B, L, H, V = 16, 2048, 1024, 32000
PAD = 0
EPS = 1e-12

NC, NS = 2, 16
NW = NC * NS
CHUNKS = (2, 6, 6, 2)
CH = 32
NBUF = 3
BL = 256


def _sc_gather(word_table, flat_ids, tok_c):
    mesh = plsc.VectorSubcoreMesh(core_axis_name="c", subcore_axis_name="s")
    b_per_w = tok_c // NW
    nch = b_per_w // CH

    @pl.kernel(out_type=jax.ShapeDtypeStruct((tok_c, H), word_table.dtype),
               mesh=mesh,
               scratch_types=(
                   [pltpu.VMEM((b_per_w,), jnp.int32)]
                   + [pltpu.VMEM((CH, H), jnp.float32)] * NBUF
                   + [pltpu.SemaphoreType.DMA] * (2 * NBUF)
               ))
    def gather_kernel(table_hbm, idx_hbm, out_hbm, idx_v, *scr):
        rows = scr[:NBUF]
        gsem = scr[NBUF:2 * NBUF]
        wsem = scr[2 * NBUF:]
        wid = jax.lax.axis_index("s") * NC + jax.lax.axis_index("c")
        base = wid * b_per_w

        pltpu.sync_copy(idx_hbm.at[pl.ds(base, b_per_w)], idx_v)

        gh = [None] * NBUF
        wh = [None] * NBUF
        for c in range(nch):
            slot = c % NBUF
            if wh[slot] is not None:
                wh[slot].wait()
            h = pltpu.make_async_copy(
                table_hbm.at[idx_v.at[pl.ds(c * CH, CH)]], rows[slot],
                gsem[slot])
            h.start()
            gh[slot] = h
            if c >= 1:
                pslot = (c - 1) % NBUF
                gh[pslot].wait()
                h = pltpu.make_async_copy(
                    rows[pslot], out_hbm.at[pl.ds(base + (c - 1) * CH, CH)],
                    wsem[pslot])
                h.start()
                wh[pslot] = h
        last = (nch - 1) % NBUF
        gh[last].wait()
        h = pltpu.make_async_copy(
            rows[last], out_hbm.at[pl.ds(base + (nch - 1) * CH, CH)],
            wsem[last])
        h.start()
        wh[last] = h
        for b in range(NBUF):
            if wh[b] is not None:
                wh[b].wait()

    return gather_kernel(word_table, flat_ids)


def _tc_chunk(k0, bc, emb, pos_emb, vectors, input_pos, gamma, beta, prev):
    def body(pos_idx_ref, emb_ref, pose_ref, vec_ref, gamma_ref, beta_ref,
             *rest):
        out_ref = rest[-1]
        lblk = pl.program_id(0)
        b = pl.program_id(1)
        x = emb_ref[0]
        row = pos_idx_ref[k0 + b] - lblk * BL
        rows = jax.lax.broadcasted_iota(jnp.int32, (BL, 1), 0)
        v = vec_ref[pl.ds(k0 + b, 1), :]
        x = jnp.where(rows == row, v, x)
        x = x + pose_ref[...]
        mean = jnp.mean(x, axis=1, keepdims=True)
        xc = x - mean
        var = jnp.mean(xc * xc, axis=1, keepdims=True)
        xhat = xc * jax.lax.rsqrt(var + EPS)
        out_ref[0] = xhat * gamma_ref[...] + beta_ref[...]

    in_specs = [
        pl.BlockSpec(memory_space=pltpu.SMEM),
        pl.BlockSpec((1, BL, H), lambda l, b: (b, l, 0)),
        pl.BlockSpec((BL, H), lambda l, b: (l, 0)),
        pl.BlockSpec(memory_space=pltpu.VMEM),
        pl.BlockSpec((1, H), lambda l, b: (0, 0)),
        pl.BlockSpec((1, H), lambda l, b: (0, 0)),
    ]
    args = [input_pos, emb, pos_emb, vectors, gamma, beta]
    kwargs = {}
    if prev is not None:
        in_specs.append(pl.BlockSpec(memory_space=pl.ANY))
        args.append(prev)
        kwargs["input_output_aliases"] = {6: 0}

    return pl.pallas_call(
        body,
        grid=(L // BL, bc),
        in_specs=in_specs,
        out_specs=pl.BlockSpec((1, BL, H),
                               lambda l, b: (k0 + b, l, 0)),
        out_shape=jax.ShapeDtypeStruct((B, L, H), jnp.float32),
        compiler_params=pltpu.CompilerParams(
            dimension_semantics=("arbitrary", "arbitrary")),
        **kwargs,
    )(*args)


@jax.jit
def kernel(input_ids, input_pos, vectors, word_table, pos_table, ln_gamma,
           ln_beta):
    flat_ids = input_ids.reshape(-1).astype(jnp.int32)
    pos_emb = jax.lax.slice(pos_table, (PAD + 1, 0), (PAD + 1 + L, H))
    input_pos = input_pos.astype(jnp.int32)
    vectors = vectors.astype(jnp.float32)
    gamma = ln_gamma.reshape(1, H)
    beta = ln_beta.reshape(1, H)

    embs = []
    k0 = 0
    for bc in CHUNKS:
        tok_c = bc * L
        ids_c = jax.lax.slice(flat_ids, (k0 * L,), (k0 * L + tok_c,))
        embs.append(_sc_gather(word_table, ids_c, tok_c))
        k0 += bc
    out = None
    k0 = 0
    for bc, emb in zip(CHUNKS, embs):
        out = _tc_chunk(k0, bc, emb.reshape(bc, L, H), pos_emb, vectors,
                        input_pos, gamma, beta, out)
        k0 += bc
    return out
